# baseline (speedup 1.0000x reference)
import functools

import jax
import jax.numpy as jnp
from jax import lax
from jax.experimental import pallas as pl
from jax.experimental.pallas import tpu as pltpu
from jax.experimental.pallas import tpu_sc as plsc

N = 10000
IN_DIM = 256
OUT_DIM = 256
HEADS = 4
E_RAW = 160000
ET = E_RAW + N
NT = 16
NC = 2
CPT = 84
CHUNK = 128
ETP = NT * CPT * CHUNK
NCH = NT * CPT
NPAD = 10112
RPT = NPAD // NT
NEG_SLOPE = 0.2
DLO = HEADS * 128

@functools.cache
def _mesh():
    return plsc.VectorSubcoreMesh(core_axis_name="c", subcore_axis_name="s",
                                  num_cores=NC, num_subcores=NT)


def _tc1_body(x_ref, wlo_ref, whi_ref, asl_ref, ash_ref, adl_ref, adh_ref,
              xps_ref, asrc_ref, adst_ref):
    xb = x_ref[...]
    lo = jnp.dot(xb, wlo_ref[...], preferred_element_type=jnp.float32)
    hi = jnp.dot(xb, whi_ref[...], preferred_element_type=jnp.float32)
    xps_ref[0] = lo
    xps_ref[1] = hi
    jj = lax.broadcasted_iota(jnp.int32, (DLO, 16), 0)
    hh = lax.broadcasted_iota(jnp.int32, (DLO, 16), 1)
    m = jnp.where((jj // 128) == hh, 1.0, 0.0).astype(jnp.float32)
    us = lo * asl_ref[...] + hi * ash_ref[...]
    ud = lo * adl_ref[...] + hi * adh_ref[...]
    asrc_ref[...] = jnp.dot(us, m, preferred_element_type=jnp.float32)
    adst_ref[...] = jnp.dot(ud, m, preferred_element_type=jnp.float32)


def _tc1(x, wlo, whi, asl, ash, adl, adh):
    blk = 1000
    grid = (N // blk,)
    return pl.pallas_call(
        _tc1_body,
        grid=grid,
        in_specs=[
            pl.BlockSpec((blk, IN_DIM), lambda i: (i, 0)),
            pl.BlockSpec((IN_DIM, DLO), lambda i: (0, 0)),
            pl.BlockSpec((IN_DIM, DLO), lambda i: (0, 0)),
            pl.BlockSpec((1, DLO), lambda i: (0, 0)),
            pl.BlockSpec((1, DLO), lambda i: (0, 0)),
            pl.BlockSpec((1, DLO), lambda i: (0, 0)),
            pl.BlockSpec((1, DLO), lambda i: (0, 0)),
        ],
        out_specs=[
            pl.BlockSpec((NC, blk, DLO), lambda i: (0, i, 0)),
            pl.BlockSpec((blk, 16), lambda i: (i, 0)),
            pl.BlockSpec((blk, 16), lambda i: (i, 0)),
        ],
        out_shape=[
            jax.ShapeDtypeStruct((NC, N, DLO), jnp.float32),
            jax.ShapeDtypeStruct((N, 16), jnp.float32),
            jax.ShapeDtypeStruct((N, 16), jnp.float32),
        ],
    )(x, wlo, whi, asl, ash, adl, adh)


NW4 = NT * 2560
SLICE4 = NW4 // NT
RED = 1280
MID = NPAD // 2
HROWS = MID // 8

def _sca_body(src1d, dst1d, asrc4, adst4, zf4, ex3, al_lo, al_hi, dnp, dnf,
              sidxv, didxv, asv, adv, dnv, tmpb, slcb, exch, alb_lo, alb_hi,
              sem):
    c = lax.axis_index("c")
    t = lax.axis_index("s")

    @pl.when(c == 0)
    def _():
        pltpu.sync_copy(asrc4, asv)
        pltpu.sync_copy(adst4, adv)
        pltpu.sync_copy(zf4.at[pl.ds(0, NW4)], dnv)

        def chunk(g, carry):
            gc = t * CPT + g
            base = gc * CHUNK
            pltpu.sync_copy(src1d.at[pl.ds(base, CHUNK)], sidxv)
            pltpu.sync_copy(dst1d.at[pl.ds(base, CHUNK)], didxv)
            for sub in range(CHUNK // 16):
                eids = lax.iota(jnp.int32, 16) + (sub * 16)
                gid = base + eids
                sv = sidxv[pl.ds(sub * 16, 16)]
                dv = didxv[pl.ds(sub * 16, 16)]
                for h in range(HEADS):
                    s = plsc.load_gather(asv, [sv * HEADS + h])
                    d = plsc.load_gather(adv, [dv * HEADS + h])
                    e = s + d
                    e = jnp.where(e >= 0.0, e, e * NEG_SLOPE)
                    ex = jnp.exp(e)
                    ex = jnp.where(gid < ET, ex, 0.0)
                    plsc.store_scatter(exch, [eids * HEADS + h], ex)
                    plsc.addupdate_scatter(dnv, [dv * HEADS + h], ex)
            pltpu.sync_copy(exch, ex3.at[pl.ds(base * HEADS, CHUNK * HEADS)])
            return carry

        lax.fori_loop(0, CPT, chunk, 0)
        pltpu.sync_copy(dnv, dnp.at[pl.ds(t * NW4, NW4)])
        plsc.subcore_barrier()
        for ss in range(SLICE4 // RED):
            off = t * SLICE4 + ss * RED
            pltpu.sync_copy(dnp.at[pl.ds(off, RED)], slcb)

            def red(tt, carry):
                pltpu.sync_copy(dnp.at[pl.ds(tt * NW4 + off, RED)], tmpb)

                def vec(i, carry2):
                    slcb[pl.ds(i * 16, 16)] = (slcb[pl.ds(i * 16, 16)]
                                               + tmpb[pl.ds(i * 16, 16)])
                    return carry2

                lax.fori_loop(0, RED // 16, vec, 0)
                return carry

            lax.fori_loop(1, NT, red, 0)
            pltpu.sync_copy(slcb, dnf.at[pl.ds(off, RED)])
        plsc.subcore_barrier()
        pltpu.sync_copy(dnf, dnv)

        def chunk2(g, carry):
            gc = t * CPT + g
            base = gc * CHUNK
            pltpu.sync_copy(dst1d.at[pl.ds(base, CHUNK)], didxv)
            pltpu.async_copy(ex3.at[pl.ds(base * HEADS, CHUNK * HEADS)],
                             exch, sem).wait()
            for sub in range(CHUNK // 16):
                eids = lax.iota(jnp.int32, 16) + (sub * 16)
                dv = didxv[pl.ds(sub * 16, 16)]
                in_lo = dv < MID
                for h in range(HEADS):
                    exv = plsc.load_gather(exch, [eids * HEADS + h])
                    dnvv = plsc.load_gather(dnv, [dv * HEADS + h])
                    al = 0.25 * exv / (dnvv + 1e-16)
                    allo = jnp.where(in_lo, al, 0.0)
                    plsc.store_scatter(alb_lo, [eids * HEADS + h], allo)
                    plsc.store_scatter(alb_hi, [eids * HEADS + h], al - allo)
            pltpu.sync_copy(alb_lo,
                            al_lo.at[pl.ds(base * HEADS, CHUNK * HEADS)])
            pltpu.sync_copy(alb_hi,
                            al_hi.at[pl.ds(base * HEADS, CHUNK * HEADS)])
            return carry

        lax.fori_loop(0, CPT, chunk2, 0)


@functools.cache
def _sca():
    return pl.kernel(
        _sca_body,
        out_type=(
            jax.ShapeDtypeStruct((ETP * HEADS,), jnp.float32),
            jax.ShapeDtypeStruct((ETP * HEADS,), jnp.float32),
            jax.ShapeDtypeStruct((ETP * HEADS,), jnp.float32),
            jax.ShapeDtypeStruct((NT * NW4,), jnp.float32),
            jax.ShapeDtypeStruct((NW4,), jnp.float32),
        ),
        mesh=_mesh(),
        compiler_params=pltpu.CompilerParams(needs_layout_passes=False),
        scratch_types=[
            pltpu.VMEM((CHUNK,), jnp.int32),
            pltpu.VMEM((CHUNK,), jnp.int32),
            pltpu.VMEM((N * HEADS,), jnp.float32),
            pltpu.VMEM((N * HEADS,), jnp.float32),
            pltpu.VMEM((NW4,), jnp.float32),
            pltpu.VMEM((RED,), jnp.float32),
            pltpu.VMEM((RED,), jnp.float32),
            pltpu.VMEM((CHUNK * HEADS,), jnp.float32),
            pltpu.VMEM((CHUNK * HEADS,), jnp.float32),
            pltpu.VMEM((CHUNK * HEADS,), jnp.float32),
            pltpu.SemaphoreType.DMA,
        ],
    )


BCH = 64
BCPT = CPT * 2
NCH64 = NT * BCPT

def _scb_body(src3, dst3, al_lo, al_hi, xps, z128, out2,
              sidxa, sidxb, didxa, didxb, didxl, alba, albb, rowsa, rowsb,
              msg, accsh, sema_r, semb_r, sema_a, semb_a):
    c = lax.axis_index("c")
    t = lax.axis_index("s")

    for p, al3 in ((0, al_lo), (1, al_hi)):
        @pl.when(t < 8)
        def _():
            pltpu.sync_copy(z128.at[pl.ds(t * HROWS, HROWS)],
                            accsh.at[pl.ds(t * HROWS, HROWS)])
        plsc.subcore_barrier()

        def load(g, sidx, didx, alb, rows, sem_r, sem_a):
            gc = t * BCPT + g
            pltpu.sync_copy(src3.at[gc], sidx)
            pltpu.sync_copy(dst3.at[gc], didx)
            pltpu.async_copy(al3.at[pl.ds(gc * BCH * HEADS, BCH * HEADS)],
                             alb, sem_a)
            pltpu.async_copy(xps.at[c].at[sidx.at[0]], rows, sem_r)

        def work(g, sidx, didx, alb, rows, sem_r, sem_a):
            gc = t * BCPT + g
            for k in range(BCH // 16):
                dv = didx[0, pl.ds(k * 16, 16)]
                didxl[pl.ds(k * 16, 16)] = jnp.clip(dv - p * MID, 0, MID - 1)
            pltpu.make_async_copy(
                al3.at[pl.ds(gc * BCH * HEADS, BCH * HEADS)], alb,
                sem_a).wait()
            pltpu.make_async_copy(
                xps.at[c].at[sidx.at[0]], rows, sem_r).wait()

            d128 = jnp.full((16,), 128, jnp.int32)

            def grp(k, carry2):
                ev = lax.iota(jnp.int32, 16) + k * 16
                ab = ev * HEADS
                a0 = plsc.load_gather(alb, [ab])
                a1 = plsc.load_gather(alb, [ab + 1])
                a2 = plsc.load_gather(alb, [ab + 2])
                a3 = plsc.load_gather(alb, [ab + 3])

                def csub(j, carry3):
                    cv = jnp.broadcast_to(j * 16, (16,))
                    for cc in range(16):
                        c0 = cv + cc
                        c1 = c0 + d128
                        c2 = c1 + d128
                        c3 = c2 + d128
                        acc = a0 * plsc.load_gather(rows, [ev, c0])
                        acc = acc + a1 * plsc.load_gather(rows, [ev, c1])
                        acc = acc + a2 * plsc.load_gather(rows, [ev, c2])
                        acc = acc + a3 * plsc.load_gather(rows, [ev, c3])
                        plsc.store_scatter(msg, [ev, c0], acc)
                    return carry3

                lax.fori_loop(0, 8, csub, 0)
                return carry2

            lax.fori_loop(0, BCH // 16, grp, 0)
            pltpu.sync_copy(msg, accsh.at[didxl], add=True)

        def chunk(g, carry):
            nxt = g + 1

            @pl.when(g % 2 == 0)
            def _():
                @pl.when(nxt < BCPT)
                def _():
                    load(nxt, sidxb, didxb, albb, rowsb, semb_r, semb_a)
                work(g, sidxa, didxa, alba, rowsa, sema_r, sema_a)

            @pl.when(g % 2 == 1)
            def _():
                @pl.when(nxt < BCPT)
                def _():
                    load(nxt, sidxa, didxa, alba, rowsa, sema_r, sema_a)
                work(g, sidxb, didxb, albb, rowsb, semb_r, semb_a)

            return carry

        load(0, sidxa, didxa, alba, rowsa, sema_r, sema_a)
        lax.fori_loop(0, BCPT, chunk, 0)
        plsc.subcore_barrier()

        @pl.when(t < 8)
        def _():
            pltpu.sync_copy(accsh.at[pl.ds(t * HROWS, HROWS)],
                            out2.at[c].at[pl.ds(p * MID + t * HROWS, HROWS)])
        plsc.subcore_barrier()


@functools.cache
def _scb():
    return pl.kernel(
        _scb_body,
        out_type=jax.ShapeDtypeStruct((NC, NPAD, 128), jnp.float32),
        mesh=_mesh(),
        compiler_params=pltpu.CompilerParams(needs_layout_passes=False),
        scratch_types=[
            pltpu.VMEM((1, BCH), jnp.int32),
            pltpu.VMEM((1, BCH), jnp.int32),
            pltpu.VMEM((1, BCH), jnp.int32),
            pltpu.VMEM((1, BCH), jnp.int32),
            pltpu.VMEM((BCH,), jnp.int32),
            pltpu.VMEM((BCH * HEADS,), jnp.float32),
            pltpu.VMEM((BCH * HEADS,), jnp.float32),
            pltpu.VMEM((BCH, DLO), jnp.float32),
            pltpu.VMEM((BCH, DLO), jnp.float32),
            pltpu.VMEM((BCH, 128), jnp.float32),
            pltpu.VMEM_SHARED((MID, 128), jnp.float32),
            pltpu.SemaphoreType.DMA,
            pltpu.SemaphoreType.DMA,
            pltpu.SemaphoreType.DMA,
            pltpu.SemaphoreType.DMA,
        ],
    )


def _tc2_body(y0_ref, y1_ref, b_ref, g_ref, be_ref, o_ref):
    y = jnp.concatenate([y0_ref[...], y1_ref[...]], axis=1) + b_ref[...]
    mu = jnp.mean(y, axis=0, keepdims=True)
    var = jnp.mean(y * y, axis=0, keepdims=True) - mu * mu
    yn = (y - mu) * lax.rsqrt(var + 1e-5) * g_ref[...] + be_ref[...]
    o_ref[...] = jnp.maximum(yn, 0.0)


def _tc2(y0, y1, b, g, be):
    return pl.pallas_call(
        _tc2_body,
        out_shape=jax.ShapeDtypeStruct((N, OUT_DIM), jnp.float32),
    )(y0, y1, b, g, be)


def kernel(x, edge_index, W, att_src, att_dst, bias, gamma, beta):
    loops = jnp.arange(N, dtype=jnp.int32)
    src = jnp.concatenate([edge_index[0].astype(jnp.int32), loops])
    dst = jnp.concatenate([edge_index[1].astype(jnp.int32), loops])
    pad = ETP - ET
    src1d = jnp.pad(src, (0, pad))
    dst1d = jnp.pad(dst, (0, pad))

    W4 = W.reshape(IN_DIM, HEADS, 2, 128)
    wlo = W4[:, :, 0, :].reshape(IN_DIM, DLO)
    whi = W4[:, :, 1, :].reshape(IN_DIM, DLO)
    asl = att_src[:, :128].reshape(1, DLO)
    ash = att_src[:, 128:].reshape(1, DLO)
    adl = att_dst[:, :128].reshape(1, DLO)
    adh = att_dst[:, 128:].reshape(1, DLO)

    xps, asrc16, adst16 = _tc1(x, wlo, whi, asl, ash, adl, adh)
    asrc4 = asrc16[:, :HEADS].reshape(N * HEADS)
    adst4 = adst16[:, :HEADS].reshape(N * HEADS)
    zf4 = jnp.zeros((NPAD * HEADS,), jnp.float32)
    ex3, al_lo, al_hi, dnp, dnf = _sca()(src1d, dst1d, asrc4, adst4, zf4)
    del ex3, dnp, dnf
    src3 = jnp.pad(src, (0, pad)).reshape(NCH64, 1, BCH)
    dst3 = jnp.pad(dst, (0, pad)).reshape(NCH64, 1, BCH)
    z128 = jnp.zeros((NPAD, 128), jnp.float32)
    out2 = _scb()(src3, dst3, al_lo, al_hi, xps, z128)
    return _tc2(out2[0, :N], out2[1, :N], bias.reshape(1, OUT_DIM),
                gamma.reshape(1, OUT_DIM), beta.reshape(1, OUT_DIM))


# speedup vs baseline: 3.7063x; 3.7063x over previous
import functools

import jax
import jax.numpy as jnp
from jax import lax
from jax.experimental import pallas as pl
from jax.experimental.pallas import tpu as pltpu
from jax.experimental.pallas import tpu_sc as plsc

N = 10000
IN_DIM = 256
OUT_DIM = 256
HEADS = 4
E_RAW = 160000
ET = E_RAW + N
NT = 16
NC = 2
CPT = 84
CHUNK = 128
ETP = NT * CPT * CHUNK
NCH = NT * CPT
NPAD = 10112
RPT = NPAD // NT
NEG_SLOPE = 0.2
DLO = HEADS * 128

@functools.cache
def _mesh():
    return plsc.VectorSubcoreMesh(core_axis_name="c", subcore_axis_name="s",
                                  num_cores=NC, num_subcores=NT)


def _tc1_body(x_ref, wlo_ref, whi_ref, asl_ref, ash_ref, adl_ref, adh_ref,
              xps_ref, asrc_ref, adst_ref):
    xb = x_ref[...]
    lo = jnp.dot(xb, wlo_ref[...], preferred_element_type=jnp.float32)
    hi = jnp.dot(xb, whi_ref[...], preferred_element_type=jnp.float32)
    xps_ref[0] = lo
    xps_ref[1] = hi
    jj = lax.broadcasted_iota(jnp.int32, (DLO, 16), 0)
    hh = lax.broadcasted_iota(jnp.int32, (DLO, 16), 1)
    m = jnp.where((jj // 128) == hh, 1.0, 0.0).astype(jnp.float32)
    us = lo * asl_ref[...] + hi * ash_ref[...]
    ud = lo * adl_ref[...] + hi * adh_ref[...]
    asrc_ref[...] = jnp.dot(us, m, preferred_element_type=jnp.float32)
    adst_ref[...] = jnp.dot(ud, m, preferred_element_type=jnp.float32)


def _tc1(x, wlo, whi, asl, ash, adl, adh):
    blk = 1000
    grid = (N // blk,)
    return pl.pallas_call(
        _tc1_body,
        grid=grid,
        in_specs=[
            pl.BlockSpec((blk, IN_DIM), lambda i: (i, 0)),
            pl.BlockSpec((IN_DIM, DLO), lambda i: (0, 0)),
            pl.BlockSpec((IN_DIM, DLO), lambda i: (0, 0)),
            pl.BlockSpec((1, DLO), lambda i: (0, 0)),
            pl.BlockSpec((1, DLO), lambda i: (0, 0)),
            pl.BlockSpec((1, DLO), lambda i: (0, 0)),
            pl.BlockSpec((1, DLO), lambda i: (0, 0)),
        ],
        out_specs=[
            pl.BlockSpec((NC, blk, DLO), lambda i: (0, i, 0)),
            pl.BlockSpec((blk, 16), lambda i: (i, 0)),
            pl.BlockSpec((blk, 16), lambda i: (i, 0)),
        ],
        out_shape=[
            jax.ShapeDtypeStruct((NC, N, DLO), jnp.float32),
            jax.ShapeDtypeStruct((N, 16), jnp.float32),
            jax.ShapeDtypeStruct((N, 16), jnp.float32),
        ],
    )(x, wlo, whi, asl, ash, adl, adh)


NW4 = NT * 2560
SLICE4 = NW4 // NT
RED = 1280
MID = NPAD // 2
HROWS = MID // 8

def _sca_body(src1d, dst1d, asrc4, adst4, zf4, ex3, al_lo, al_hi, dnp, dnf,
              sidxv, didxv, asv, adv, dnv, tmpb, slcb, exch, alb_lo, alb_hi,
              sem):
    c = lax.axis_index("c")
    t = lax.axis_index("s")

    @pl.when(c == 0)
    def _():
        pltpu.sync_copy(asrc4, asv)
        pltpu.sync_copy(adst4, adv)
        pltpu.sync_copy(zf4.at[pl.ds(0, NW4)], dnv)

        def chunk(g, carry):
            gc = t * CPT + g
            base = gc * CHUNK
            pltpu.sync_copy(src1d.at[pl.ds(base, CHUNK)], sidxv)
            pltpu.sync_copy(dst1d.at[pl.ds(base, CHUNK)], didxv)
            for sub in range(CHUNK // 16):
                eids = lax.iota(jnp.int32, 16) + (sub * 16)
                gid = base + eids
                sv = sidxv[pl.ds(sub * 16, 16)]
                dv = didxv[pl.ds(sub * 16, 16)]
                for h in range(HEADS):
                    s = plsc.load_gather(asv, [sv * HEADS + h])
                    d = plsc.load_gather(adv, [dv * HEADS + h])
                    e = s + d
                    e = jnp.where(e >= 0.0, e, e * NEG_SLOPE)
                    ex = jnp.exp(e)
                    ex = jnp.where(gid < ET, ex, 0.0)
                    plsc.store_scatter(exch, [eids * HEADS + h], ex)
                    plsc.addupdate_scatter(dnv, [dv * HEADS + h], ex)
            pltpu.sync_copy(exch, ex3.at[pl.ds(base * HEADS, CHUNK * HEADS)])
            return carry

        lax.fori_loop(0, CPT, chunk, 0)
        pltpu.sync_copy(dnv, dnp.at[pl.ds(t * NW4, NW4)])
        plsc.subcore_barrier()
        for ss in range(SLICE4 // RED):
            off = t * SLICE4 + ss * RED
            pltpu.sync_copy(dnp.at[pl.ds(off, RED)], slcb)

            def red(tt, carry):
                pltpu.sync_copy(dnp.at[pl.ds(tt * NW4 + off, RED)], tmpb)

                def vec(i, carry2):
                    slcb[pl.ds(i * 16, 16)] = (slcb[pl.ds(i * 16, 16)]
                                               + tmpb[pl.ds(i * 16, 16)])
                    return carry2

                lax.fori_loop(0, RED // 16, vec, 0)
                return carry

            lax.fori_loop(1, NT, red, 0)
            pltpu.sync_copy(slcb, dnf.at[pl.ds(off, RED)])
        plsc.subcore_barrier()
        pltpu.sync_copy(dnf, dnv)

        def chunk2(g, carry):
            gc = t * CPT + g
            base = gc * CHUNK
            pltpu.sync_copy(dst1d.at[pl.ds(base, CHUNK)], didxv)
            pltpu.async_copy(ex3.at[pl.ds(base * HEADS, CHUNK * HEADS)],
                             exch, sem).wait()
            for sub in range(CHUNK // 16):
                eids = lax.iota(jnp.int32, 16) + (sub * 16)
                dv = didxv[pl.ds(sub * 16, 16)]
                in_lo = dv < MID
                for h in range(HEADS):
                    exv = plsc.load_gather(exch, [eids * HEADS + h])
                    dnvv = plsc.load_gather(dnv, [dv * HEADS + h])
                    al = 0.25 * exv / (dnvv + 1e-16)
                    allo = jnp.where(in_lo, al, 0.0)
                    plsc.store_scatter(alb_lo, [eids * HEADS + h], allo)
                    plsc.store_scatter(alb_hi, [eids * HEADS + h], al - allo)
            pltpu.sync_copy(alb_lo,
                            al_lo.at[pl.ds(base * HEADS, CHUNK * HEADS)])
            pltpu.sync_copy(alb_hi,
                            al_hi.at[pl.ds(base * HEADS, CHUNK * HEADS)])
            return carry

        lax.fori_loop(0, CPT, chunk2, 0)


@functools.cache
def _sca():
    return pl.kernel(
        _sca_body,
        out_type=(
            jax.ShapeDtypeStruct((ETP * HEADS,), jnp.float32),
            jax.ShapeDtypeStruct((ETP * HEADS,), jnp.float32),
            jax.ShapeDtypeStruct((ETP * HEADS,), jnp.float32),
            jax.ShapeDtypeStruct((NT * NW4,), jnp.float32),
            jax.ShapeDtypeStruct((NW4,), jnp.float32),
        ),
        mesh=_mesh(),
        compiler_params=pltpu.CompilerParams(needs_layout_passes=False),
        scratch_types=[
            pltpu.VMEM((CHUNK,), jnp.int32),
            pltpu.VMEM((CHUNK,), jnp.int32),
            pltpu.VMEM((N * HEADS,), jnp.float32),
            pltpu.VMEM((N * HEADS,), jnp.float32),
            pltpu.VMEM((NW4,), jnp.float32),
            pltpu.VMEM((RED,), jnp.float32),
            pltpu.VMEM((RED,), jnp.float32),
            pltpu.VMEM((CHUNK * HEADS,), jnp.float32),
            pltpu.VMEM((CHUNK * HEADS,), jnp.float32),
            pltpu.VMEM((CHUNK * HEADS,), jnp.float32),
            pltpu.SemaphoreType.DMA,
        ],
    )


def _lane_splat(vec, lane):
    idx = jnp.full((16, 1), lane, jnp.int32)
    return lax.gather(
        vec, idx,
        lax.GatherDimensionNumbers(offset_dims=(), collapsed_slice_dims=(0,),
                                   start_index_map=(0,)),
        (1,), mode=lax.GatherScatterMode.PROMISE_IN_BOUNDS)


BCH = 64
BCPT = CPT * 2
NCH64 = NT * BCPT

def _scb_body(src3, dst3, al_lo, al_hi, xps, z128, out2,
              sidxa, sidxb, didxa, didxb, didxl, alba, albb, rowsa, rowsb,
              msg, accsh, sema_r, semb_r, sema_a, semb_a):
    c = lax.axis_index("c")
    t = lax.axis_index("s")

    for p, al3 in ((0, al_lo), (1, al_hi)):
        @pl.when(t < 8)
        def _():
            pltpu.sync_copy(z128.at[pl.ds(t * HROWS, HROWS)],
                            accsh.at[pl.ds(t * HROWS, HROWS)])
        plsc.subcore_barrier()

        def load(g, sidx, didx, alb, rows, sem_r, sem_a):
            gc = t * BCPT + g
            pltpu.sync_copy(src3.at[gc], sidx)
            pltpu.sync_copy(dst3.at[gc], didx)
            pltpu.async_copy(al3.at[pl.ds(gc * BCH * HEADS, BCH * HEADS)],
                             alb, sem_a)
            pltpu.async_copy(xps.at[c].at[sidx.at[0]], rows, sem_r)

        def work(g, sidx, didx, alb, rows, sem_r, sem_a):
            gc = t * BCPT + g
            for k in range(BCH // 16):
                dv = didx[0, pl.ds(k * 16, 16)]
                didxl[pl.ds(k * 16, 16)] = jnp.clip(dv - p * MID, 0, MID - 1)
            pltpu.make_async_copy(
                al3.at[pl.ds(gc * BCH * HEADS, BCH * HEADS)], alb,
                sem_a).wait()
            pltpu.make_async_copy(
                xps.at[c].at[sidx.at[0]], rows, sem_r).wait()

            def grp(k, carry2):
                ev = lax.iota(jnp.int32, 16) + k * 16
                ab = ev * HEADS
                av0 = plsc.load_gather(alb, [ab])
                av1 = plsc.load_gather(alb, [ab + 1])
                av2 = plsc.load_gather(alb, [ab + 2])
                av3 = plsc.load_gather(alb, [ab + 3])
                for l in range(16):
                    e = k * 16 + l
                    a0 = _lane_splat(av0, l)
                    a1 = _lane_splat(av1, l)
                    a2 = _lane_splat(av2, l)
                    a3 = _lane_splat(av3, l)
                    for v in range(8):
                        mv = (a0 * rows[e, pl.ds(v * 16, 16)]
                              + a1 * rows[e, pl.ds(128 + v * 16, 16)]
                              + a2 * rows[e, pl.ds(256 + v * 16, 16)]
                              + a3 * rows[e, pl.ds(384 + v * 16, 16)])
                        msg[e, pl.ds(v * 16, 16)] = mv
                return carry2

            lax.fori_loop(0, BCH // 16, grp, 0)
            pltpu.sync_copy(msg, accsh.at[didxl], add=True)

        def chunk(g, carry):
            nxt = g + 1

            @pl.when(g % 2 == 0)
            def _():
                @pl.when(nxt < BCPT)
                def _():
                    load(nxt, sidxb, didxb, albb, rowsb, semb_r, semb_a)
                work(g, sidxa, didxa, alba, rowsa, sema_r, sema_a)

            @pl.when(g % 2 == 1)
            def _():
                @pl.when(nxt < BCPT)
                def _():
                    load(nxt, sidxa, didxa, alba, rowsa, sema_r, sema_a)
                work(g, sidxb, didxb, albb, rowsb, semb_r, semb_a)

            return carry

        load(0, sidxa, didxa, alba, rowsa, sema_r, sema_a)
        lax.fori_loop(0, BCPT, chunk, 0)
        plsc.subcore_barrier()

        @pl.when(t < 8)
        def _():
            pltpu.sync_copy(accsh.at[pl.ds(t * HROWS, HROWS)],
                            out2.at[c].at[pl.ds(p * MID + t * HROWS, HROWS)])
        plsc.subcore_barrier()


@functools.cache
def _scb():
    return pl.kernel(
        _scb_body,
        out_type=jax.ShapeDtypeStruct((NC, NPAD, 128), jnp.float32),
        mesh=_mesh(),
        compiler_params=pltpu.CompilerParams(needs_layout_passes=False),
        scratch_types=[
            pltpu.VMEM((1, BCH), jnp.int32),
            pltpu.VMEM((1, BCH), jnp.int32),
            pltpu.VMEM((1, BCH), jnp.int32),
            pltpu.VMEM((1, BCH), jnp.int32),
            pltpu.VMEM((BCH,), jnp.int32),
            pltpu.VMEM((BCH * HEADS,), jnp.float32),
            pltpu.VMEM((BCH * HEADS,), jnp.float32),
            pltpu.VMEM((BCH, DLO), jnp.float32),
            pltpu.VMEM((BCH, DLO), jnp.float32),
            pltpu.VMEM((BCH, 128), jnp.float32),
            pltpu.VMEM_SHARED((MID, 128), jnp.float32),
            pltpu.SemaphoreType.DMA,
            pltpu.SemaphoreType.DMA,
            pltpu.SemaphoreType.DMA,
            pltpu.SemaphoreType.DMA,
        ],
    )


def _tc2_body(y0_ref, y1_ref, b_ref, g_ref, be_ref, o_ref):
    y = jnp.concatenate([y0_ref[...], y1_ref[...]], axis=1) + b_ref[...]
    mu = jnp.mean(y, axis=0, keepdims=True)
    var = jnp.mean(y * y, axis=0, keepdims=True) - mu * mu
    yn = (y - mu) * lax.rsqrt(var + 1e-5) * g_ref[...] + be_ref[...]
    o_ref[...] = jnp.maximum(yn, 0.0)


def _tc2(y0, y1, b, g, be):
    return pl.pallas_call(
        _tc2_body,
        out_shape=jax.ShapeDtypeStruct((N, OUT_DIM), jnp.float32),
    )(y0, y1, b, g, be)


def kernel(x, edge_index, W, att_src, att_dst, bias, gamma, beta):
    loops = jnp.arange(N, dtype=jnp.int32)
    src = jnp.concatenate([edge_index[0].astype(jnp.int32), loops])
    dst = jnp.concatenate([edge_index[1].astype(jnp.int32), loops])
    pad = ETP - ET
    src1d = jnp.pad(src, (0, pad))
    dst1d = jnp.pad(dst, (0, pad))

    W4 = W.reshape(IN_DIM, HEADS, 2, 128)
    wlo = W4[:, :, 0, :].reshape(IN_DIM, DLO)
    whi = W4[:, :, 1, :].reshape(IN_DIM, DLO)
    asl = att_src[:, :128].reshape(1, DLO)
    ash = att_src[:, 128:].reshape(1, DLO)
    adl = att_dst[:, :128].reshape(1, DLO)
    adh = att_dst[:, 128:].reshape(1, DLO)

    xps, asrc16, adst16 = _tc1(x, wlo, whi, asl, ash, adl, adh)
    asrc4 = asrc16[:, :HEADS].reshape(N * HEADS)
    adst4 = adst16[:, :HEADS].reshape(N * HEADS)
    zf4 = jnp.zeros((NPAD * HEADS,), jnp.float32)
    ex3, al_lo, al_hi, dnp, dnf = _sca()(src1d, dst1d, asrc4, adst4, zf4)
    del ex3, dnp, dnf
    src3 = jnp.pad(src, (0, pad)).reshape(NCH64, 1, BCH)
    dst3 = jnp.pad(dst, (0, pad)).reshape(NCH64, 1, BCH)
    z128 = jnp.zeros((NPAD, 128), jnp.float32)
    out2 = _scb()(src3, dst3, al_lo, al_hi, xps, z128)
    return _tc2(out2[0, :N], out2[1, :N], bias.reshape(1, OUT_DIM),
                gamma.reshape(1, OUT_DIM), beta.reshape(1, OUT_DIM))


# speedup vs baseline: 4.0197x; 1.0846x over previous
import functools

import jax
import jax.numpy as jnp
from jax import lax
from jax.experimental import pallas as pl
from jax.experimental.pallas import tpu as pltpu
from jax.experimental.pallas import tpu_sc as plsc

N = 10000
IN_DIM = 256
OUT_DIM = 256
HEADS = 4
E_RAW = 160000
ET = E_RAW + N
NT = 16
NC = 2
CPT = 84
CHUNK = 128
ETP = NT * CPT * CHUNK
NCH = NT * CPT
NPAD = 10112
RPT = NPAD // NT
NEG_SLOPE = 0.2
DLO = HEADS * 128

@functools.cache
def _mesh():
    return plsc.VectorSubcoreMesh(core_axis_name="c", subcore_axis_name="s",
                                  num_cores=NC, num_subcores=NT)


def _tc1_body(x_ref, wlo_ref, whi_ref, asl_ref, ash_ref, adl_ref, adh_ref,
              xps_ref, asrc_ref, adst_ref):
    xb = x_ref[...]
    lo = jnp.dot(xb, wlo_ref[...], preferred_element_type=jnp.float32)
    hi = jnp.dot(xb, whi_ref[...], preferred_element_type=jnp.float32)
    xps_ref[0] = lo
    xps_ref[1] = hi
    jj = lax.broadcasted_iota(jnp.int32, (DLO, 16), 0)
    hh = lax.broadcasted_iota(jnp.int32, (DLO, 16), 1)
    m = jnp.where((jj // 128) == hh, 1.0, 0.0).astype(jnp.float32)
    us = lo * asl_ref[...] + hi * ash_ref[...]
    ud = lo * adl_ref[...] + hi * adh_ref[...]
    asrc_ref[...] = jnp.dot(us, m, preferred_element_type=jnp.float32)
    adst_ref[...] = jnp.dot(ud, m, preferred_element_type=jnp.float32)


def _tc1(x, wlo, whi, asl, ash, adl, adh):
    blk = 1000
    grid = (N // blk,)
    return pl.pallas_call(
        _tc1_body,
        grid=grid,
        in_specs=[
            pl.BlockSpec((blk, IN_DIM), lambda i: (i, 0)),
            pl.BlockSpec((IN_DIM, DLO), lambda i: (0, 0)),
            pl.BlockSpec((IN_DIM, DLO), lambda i: (0, 0)),
            pl.BlockSpec((1, DLO), lambda i: (0, 0)),
            pl.BlockSpec((1, DLO), lambda i: (0, 0)),
            pl.BlockSpec((1, DLO), lambda i: (0, 0)),
            pl.BlockSpec((1, DLO), lambda i: (0, 0)),
        ],
        out_specs=[
            pl.BlockSpec((NC, blk, DLO), lambda i: (0, i, 0)),
            pl.BlockSpec((blk, 16), lambda i: (i, 0)),
            pl.BlockSpec((blk, 16), lambda i: (i, 0)),
        ],
        out_shape=[
            jax.ShapeDtypeStruct((NC, N, DLO), jnp.float32),
            jax.ShapeDtypeStruct((N, 16), jnp.float32),
            jax.ShapeDtypeStruct((N, 16), jnp.float32),
        ],
    )(x, wlo, whi, asl, ash, adl, adh)


NW4 = NT * 2560
SLICE4 = NW4 // NT
RED = 1280
MID = NPAD // 2
HROWS = MID // 8

def _sca_body(src1d, dst1d, asrc4, adst4, zf4, ex3, al_lo, al_hi, dnp, dnf,
              sidxv, didxv, asv, adv, dnv, tmpb, slcb, exch, alb_lo, alb_hi,
              sem):
    c = lax.axis_index("c")
    t = lax.axis_index("s")

    @pl.when(c == 0)
    def _():
        pltpu.sync_copy(asrc4, asv)
        pltpu.sync_copy(adst4, adv)
        pltpu.sync_copy(zf4.at[pl.ds(0, NW4)], dnv)

        def chunk(g, carry):
            gc = t * CPT + g
            base = gc * CHUNK
            pltpu.sync_copy(src1d.at[pl.ds(base, CHUNK)], sidxv)
            pltpu.sync_copy(dst1d.at[pl.ds(base, CHUNK)], didxv)
            for sub in range(CHUNK // 16):
                eids = lax.iota(jnp.int32, 16) + (sub * 16)
                gid = base + eids
                sv = sidxv[pl.ds(sub * 16, 16)]
                dv = didxv[pl.ds(sub * 16, 16)]
                for h in range(HEADS):
                    s = plsc.load_gather(asv, [sv * HEADS + h])
                    d = plsc.load_gather(adv, [dv * HEADS + h])
                    e = s + d
                    e = jnp.where(e >= 0.0, e, e * NEG_SLOPE)
                    ex = jnp.exp(e)
                    ex = jnp.where(gid < ET, ex, 0.0)
                    plsc.store_scatter(exch, [eids * HEADS + h], ex)
                    plsc.addupdate_scatter(dnv, [dv * HEADS + h], ex)
            pltpu.sync_copy(exch, ex3.at[pl.ds(base * HEADS, CHUNK * HEADS)])
            return carry

        lax.fori_loop(0, CPT, chunk, 0)
        pltpu.sync_copy(dnv, dnp.at[pl.ds(t * NW4, NW4)])
        plsc.subcore_barrier()
        for ss in range(SLICE4 // RED):
            off = t * SLICE4 + ss * RED
            pltpu.sync_copy(dnp.at[pl.ds(off, RED)], slcb)

            def red(tt, carry):
                pltpu.sync_copy(dnp.at[pl.ds(tt * NW4 + off, RED)], tmpb)

                def vec(i, carry2):
                    slcb[pl.ds(i * 16, 16)] = (slcb[pl.ds(i * 16, 16)]
                                               + tmpb[pl.ds(i * 16, 16)])
                    return carry2

                lax.fori_loop(0, RED // 16, vec, 0)
                return carry

            lax.fori_loop(1, NT, red, 0)
            pltpu.sync_copy(slcb, dnf.at[pl.ds(off, RED)])
        plsc.subcore_barrier()
        pltpu.sync_copy(dnf, dnv)

        def chunk2(g, carry):
            gc = t * CPT + g
            base = gc * CHUNK
            pltpu.sync_copy(dst1d.at[pl.ds(base, CHUNK)], didxv)
            pltpu.async_copy(ex3.at[pl.ds(base * HEADS, CHUNK * HEADS)],
                             exch, sem).wait()
            for sub in range(CHUNK // 16):
                eids = lax.iota(jnp.int32, 16) + (sub * 16)
                dv = didxv[pl.ds(sub * 16, 16)]
                in_lo = dv < MID
                for h in range(HEADS):
                    exv = plsc.load_gather(exch, [eids * HEADS + h])
                    dnvv = plsc.load_gather(dnv, [dv * HEADS + h])
                    al = 0.25 * exv / (dnvv + 1e-16)
                    allo = jnp.where(in_lo, al, 0.0)
                    plsc.store_scatter(alb_lo, [eids * HEADS + h], allo)
                    plsc.store_scatter(alb_hi, [eids * HEADS + h], al - allo)
            pltpu.sync_copy(alb_lo,
                            al_lo.at[pl.ds(base * HEADS, CHUNK * HEADS)])
            pltpu.sync_copy(alb_hi,
                            al_hi.at[pl.ds(base * HEADS, CHUNK * HEADS)])
            return carry

        lax.fori_loop(0, CPT, chunk2, 0)


@functools.cache
def _sca():
    return pl.kernel(
        _sca_body,
        out_type=(
            jax.ShapeDtypeStruct((ETP * HEADS,), jnp.float32),
            jax.ShapeDtypeStruct((ETP * HEADS,), jnp.float32),
            jax.ShapeDtypeStruct((ETP * HEADS,), jnp.float32),
            jax.ShapeDtypeStruct((NT * NW4,), jnp.float32),
            jax.ShapeDtypeStruct((NW4,), jnp.float32),
        ),
        mesh=_mesh(),
        compiler_params=pltpu.CompilerParams(needs_layout_passes=False),
        scratch_types=[
            pltpu.VMEM((CHUNK,), jnp.int32),
            pltpu.VMEM((CHUNK,), jnp.int32),
            pltpu.VMEM((N * HEADS,), jnp.float32),
            pltpu.VMEM((N * HEADS,), jnp.float32),
            pltpu.VMEM((NW4,), jnp.float32),
            pltpu.VMEM((RED,), jnp.float32),
            pltpu.VMEM((RED,), jnp.float32),
            pltpu.VMEM((CHUNK * HEADS,), jnp.float32),
            pltpu.VMEM((CHUNK * HEADS,), jnp.float32),
            pltpu.VMEM((CHUNK * HEADS,), jnp.float32),
            pltpu.SemaphoreType.DMA,
        ],
    )


def _lane_splat(vec, lane):
    idx = jnp.full((16, 1), lane, jnp.int32)
    return lax.gather(
        vec, idx,
        lax.GatherDimensionNumbers(offset_dims=(), collapsed_slice_dims=(0,),
                                   start_index_map=(0,)),
        (1,), mode=lax.GatherScatterMode.PROMISE_IN_BOUNDS)


BCH = 64
BCPT = CPT * 2
NCH64 = NT * BCPT

def _scb_body(src3, dst3, al_lo, al_hi, xps, z128, out2,
              sidxa, sidxb, didxa, didxb, didxl, alba, albb, rowsa, rowsb,
              msg, accsh, sema_r, semb_r, sema_a, semb_a,
              sema_s, semb_s, sema_d, semb_d):
    c = lax.axis_index("c")
    t = lax.axis_index("s")

    for p, al3 in ((0, al_lo), (1, al_hi)):
        @pl.when(t < 8)
        def _():
            pltpu.sync_copy(z128.at[pl.ds(t * HROWS, HROWS)],
                            accsh.at[pl.ds(t * HROWS, HROWS)])
        plsc.subcore_barrier()

        def idx_load(g, sidx, didx, sem_s, sem_d):
            gc = t * BCPT + g
            pltpu.async_copy(src3.at[gc], sidx, sem_s)
            pltpu.async_copy(dst3.at[gc], didx, sem_d)

        def idx_wait(g, sidx, didx, sem_s, sem_d):
            gc = t * BCPT + g
            pltpu.make_async_copy(src3.at[gc], sidx, sem_s).wait()
            pltpu.make_async_copy(dst3.at[gc], didx, sem_d).wait()

        def rows_load(g, sidx, alb, rows, sem_r, sem_a):
            gc = t * BCPT + g
            pltpu.async_copy(al3.at[pl.ds(gc * BCH * HEADS, BCH * HEADS)],
                             alb, sem_a)
            pltpu.async_copy(xps.at[c].at[sidx.at[0]], rows, sem_r)

        def rows_wait(g, sidx, alb, rows, sem_r, sem_a):
            gc = t * BCPT + g
            pltpu.make_async_copy(
                al3.at[pl.ds(gc * BCH * HEADS, BCH * HEADS)], alb,
                sem_a).wait()
            pltpu.make_async_copy(
                xps.at[c].at[sidx.at[0]], rows, sem_r).wait()

        def make_didxl(didx):
            for k in range(BCH // 16):
                dv = didx[0, pl.ds(k * 16, 16)]
                didxl[pl.ds(k * 16, 16)] = jnp.clip(dv - p * MID, 0, MID - 1)

        def compute(alb, rows):
            def grp(k, carry2):
                ev = lax.iota(jnp.int32, 16) + k * 16
                ab = ev * HEADS
                av0 = plsc.load_gather(alb, [ab])
                av1 = plsc.load_gather(alb, [ab + 1])
                av2 = plsc.load_gather(alb, [ab + 2])
                av3 = plsc.load_gather(alb, [ab + 3])
                for l in range(16):
                    e = k * 16 + l
                    a0 = _lane_splat(av0, l)
                    a1 = _lane_splat(av1, l)
                    a2 = _lane_splat(av2, l)
                    a3 = _lane_splat(av3, l)
                    for v in range(8):
                        mv = (a0 * rows[e, pl.ds(v * 16, 16)]
                              + a1 * rows[e, pl.ds(128 + v * 16, 16)]
                              + a2 * rows[e, pl.ds(256 + v * 16, 16)]
                              + a3 * rows[e, pl.ds(384 + v * 16, 16)])
                        msg[e, pl.ds(v * 16, 16)] = mv
                return carry2

            lax.fori_loop(0, BCH // 16, grp, 0)
            pltpu.sync_copy(msg, accsh.at[didxl], add=True)

        bufA = (sidxa, didxa, alba, rowsa, sema_r, sema_a, sema_s, sema_d)
        bufB = (sidxb, didxb, albb, rowsb, semb_r, semb_a, semb_s, semb_d)

        def step(g, cur, nxt):
            (csi, cdi, cal, crw, csr, csa, css, csd) = cur
            (nsi, ndi, nal, nrw, nsr, nsa, nss, nsd) = nxt
            rows_wait(g, csi, cal, crw, csr, csa)
            make_didxl(cdi)

            @pl.when(g + 2 < BCPT)
            def _():
                idx_load(g + 2, csi, cdi, css, csd)

            @pl.when(g + 1 < BCPT)
            def _():
                idx_wait(g + 1, nsi, ndi, nss, nsd)
                rows_load(g + 1, nsi, nal, nrw, nsr, nsa)

            compute(cal, crw)

        def chunk(g, carry):
            @pl.when(g % 2 == 0)
            def _():
                step(g, bufA, bufB)

            @pl.when(g % 2 == 1)
            def _():
                step(g, bufB, bufA)

            return carry

        idx_load(0, sidxa, didxa, sema_s, sema_d)
        idx_load(1, sidxb, didxb, semb_s, semb_d)
        idx_wait(0, sidxa, didxa, sema_s, sema_d)
        rows_load(0, sidxa, alba, rowsa, sema_r, sema_a)
        lax.fori_loop(0, BCPT, chunk, 0)
        plsc.subcore_barrier()

        @pl.when(t < 8)
        def _():
            pltpu.sync_copy(accsh.at[pl.ds(t * HROWS, HROWS)],
                            out2.at[c].at[pl.ds(p * MID + t * HROWS, HROWS)])
        plsc.subcore_barrier()


@functools.cache
def _scb():
    return pl.kernel(
        _scb_body,
        out_type=jax.ShapeDtypeStruct((NC, NPAD, 128), jnp.float32),
        mesh=_mesh(),
        compiler_params=pltpu.CompilerParams(needs_layout_passes=False),
        scratch_types=[
            pltpu.VMEM((1, BCH), jnp.int32),
            pltpu.VMEM((1, BCH), jnp.int32),
            pltpu.VMEM((1, BCH), jnp.int32),
            pltpu.VMEM((1, BCH), jnp.int32),
            pltpu.VMEM((BCH,), jnp.int32),
            pltpu.VMEM((BCH * HEADS,), jnp.float32),
            pltpu.VMEM((BCH * HEADS,), jnp.float32),
            pltpu.VMEM((BCH, DLO), jnp.float32),
            pltpu.VMEM((BCH, DLO), jnp.float32),
            pltpu.VMEM((BCH, 128), jnp.float32),
            pltpu.VMEM_SHARED((MID, 128), jnp.float32),
            pltpu.SemaphoreType.DMA,
            pltpu.SemaphoreType.DMA,
            pltpu.SemaphoreType.DMA,
            pltpu.SemaphoreType.DMA,
            pltpu.SemaphoreType.DMA,
            pltpu.SemaphoreType.DMA,
            pltpu.SemaphoreType.DMA,
            pltpu.SemaphoreType.DMA,
        ],
    )


def _tc2_body(y0_ref, y1_ref, b_ref, g_ref, be_ref, o_ref):
    y = jnp.concatenate([y0_ref[...], y1_ref[...]], axis=1) + b_ref[...]
    mu = jnp.mean(y, axis=0, keepdims=True)
    var = jnp.mean(y * y, axis=0, keepdims=True) - mu * mu
    yn = (y - mu) * lax.rsqrt(var + 1e-5) * g_ref[...] + be_ref[...]
    o_ref[...] = jnp.maximum(yn, 0.0)


def _tc2(y0, y1, b, g, be):
    return pl.pallas_call(
        _tc2_body,
        out_shape=jax.ShapeDtypeStruct((N, OUT_DIM), jnp.float32),
    )(y0, y1, b, g, be)


def kernel(x, edge_index, W, att_src, att_dst, bias, gamma, beta):
    loops = jnp.arange(N, dtype=jnp.int32)
    src = jnp.concatenate([edge_index[0].astype(jnp.int32), loops])
    dst = jnp.concatenate([edge_index[1].astype(jnp.int32), loops])
    pad = ETP - ET
    src1d = jnp.pad(src, (0, pad))
    dst1d = jnp.pad(dst, (0, pad))

    W4 = W.reshape(IN_DIM, HEADS, 2, 128)
    wlo = W4[:, :, 0, :].reshape(IN_DIM, DLO)
    whi = W4[:, :, 1, :].reshape(IN_DIM, DLO)
    asl = att_src[:, :128].reshape(1, DLO)
    ash = att_src[:, 128:].reshape(1, DLO)
    adl = att_dst[:, :128].reshape(1, DLO)
    adh = att_dst[:, 128:].reshape(1, DLO)

    xps, asrc16, adst16 = _tc1(x, wlo, whi, asl, ash, adl, adh)
    asrc4 = asrc16[:, :HEADS].reshape(N * HEADS)
    adst4 = adst16[:, :HEADS].reshape(N * HEADS)
    zf4 = jnp.zeros((NPAD * HEADS,), jnp.float32)
    ex3, al_lo, al_hi, dnp, dnf = _sca()(src1d, dst1d, asrc4, adst4, zf4)
    del ex3, dnp, dnf
    src3 = jnp.pad(src, (0, pad)).reshape(NCH64, 1, BCH)
    dst3 = jnp.pad(dst, (0, pad)).reshape(NCH64, 1, BCH)
    z128 = jnp.zeros((NPAD, 128), jnp.float32)
    out2 = _scb()(src3, dst3, al_lo, al_hi, xps, z128)
    return _tc2(out2[0, :N], out2[1, :N], bias.reshape(1, OUT_DIM),
                gamma.reshape(1, OUT_DIM), beta.reshape(1, OUT_DIM))


# speedup vs baseline: 4.4559x; 1.1085x over previous
import functools

import jax
import jax.numpy as jnp
from jax import lax
from jax.experimental import pallas as pl
from jax.experimental.pallas import tpu as pltpu
from jax.experimental.pallas import tpu_sc as plsc

N = 10000
IN_DIM = 256
OUT_DIM = 256
HEADS = 4
E_RAW = 160000
ET = E_RAW + N
NT = 16
NC = 2
CPT = 84
CHUNK = 128
ETP = NT * CPT * CHUNK
NCH = NT * CPT
NPAD = 10112
RPT = NPAD // NT
NEG_SLOPE = 0.2
DLO = HEADS * 128

@functools.cache
def _mesh():
    return plsc.VectorSubcoreMesh(core_axis_name="c", subcore_axis_name="s",
                                  num_cores=NC, num_subcores=NT)


def _tc1_body(x_ref, wlo_ref, whi_ref, asl_ref, ash_ref, adl_ref, adh_ref,
              xps_ref, asrc_ref, adst_ref):
    xb = x_ref[...]
    lo = jnp.dot(xb, wlo_ref[...], preferred_element_type=jnp.float32)
    hi = jnp.dot(xb, whi_ref[...], preferred_element_type=jnp.float32)
    xps_ref[0] = lo
    xps_ref[1] = hi
    jj = lax.broadcasted_iota(jnp.int32, (DLO, 16), 0)
    hh = lax.broadcasted_iota(jnp.int32, (DLO, 16), 1)
    m = jnp.where((jj // 128) == hh, 1.0, 0.0).astype(jnp.float32)
    us = lo * asl_ref[...] + hi * ash_ref[...]
    ud = lo * adl_ref[...] + hi * adh_ref[...]
    asrc_ref[...] = jnp.dot(us, m, preferred_element_type=jnp.float32)
    adst_ref[...] = jnp.dot(ud, m, preferred_element_type=jnp.float32)


def _tc1(x, wlo, whi, asl, ash, adl, adh):
    blk = 1000
    grid = (N // blk,)
    return pl.pallas_call(
        _tc1_body,
        grid=grid,
        in_specs=[
            pl.BlockSpec((blk, IN_DIM), lambda i: (i, 0)),
            pl.BlockSpec((IN_DIM, DLO), lambda i: (0, 0)),
            pl.BlockSpec((IN_DIM, DLO), lambda i: (0, 0)),
            pl.BlockSpec((1, DLO), lambda i: (0, 0)),
            pl.BlockSpec((1, DLO), lambda i: (0, 0)),
            pl.BlockSpec((1, DLO), lambda i: (0, 0)),
            pl.BlockSpec((1, DLO), lambda i: (0, 0)),
        ],
        out_specs=[
            pl.BlockSpec((NC, blk, DLO), lambda i: (0, i, 0)),
            pl.BlockSpec((blk, 16), lambda i: (i, 0)),
            pl.BlockSpec((blk, 16), lambda i: (i, 0)),
        ],
        out_shape=[
            jax.ShapeDtypeStruct((NC, N, DLO), jnp.float32),
            jax.ShapeDtypeStruct((N, 16), jnp.float32),
            jax.ShapeDtypeStruct((N, 16), jnp.float32),
        ],
    )(x, wlo, whi, asl, ash, adl, adh)


NW4 = NT * 2560
SLICE4 = NW4 // NT
RED = 1280
MID = NPAD // 2
HROWS = MID // 8

def _sca_body(src1d, dst1d, asrc4, adst4, zf4, ex3, al_lo, al_hi, dnp, dnf,
              sidxv, didxv, asv, adv, dnv, tmpb, slcb, exch, alb_lo, alb_hi,
              sem):
    c = lax.axis_index("c")
    t = lax.axis_index("s")

    @pl.when(c == 0)
    def _():
        pltpu.sync_copy(asrc4, asv)
        pltpu.sync_copy(adst4, adv)
        pltpu.sync_copy(zf4.at[pl.ds(0, NW4)], dnv)

        def chunk(g, carry):
            gc = t * CPT + g
            base = gc * CHUNK
            pltpu.sync_copy(src1d.at[pl.ds(base, CHUNK)], sidxv)
            pltpu.sync_copy(dst1d.at[pl.ds(base, CHUNK)], didxv)
            for sub in range(CHUNK // 16):
                eids = lax.iota(jnp.int32, 16) + (sub * 16)
                gid = base + eids
                sv = sidxv[pl.ds(sub * 16, 16)]
                dv = didxv[pl.ds(sub * 16, 16)]
                for h in range(HEADS):
                    s = plsc.load_gather(asv, [sv * HEADS + h])
                    d = plsc.load_gather(adv, [dv * HEADS + h])
                    e = s + d
                    e = jnp.where(e >= 0.0, e, e * NEG_SLOPE)
                    ex = jnp.exp(e)
                    ex = jnp.where(gid < ET, ex, 0.0)
                    plsc.store_scatter(exch, [eids * HEADS + h], ex)
                    plsc.addupdate_scatter(dnv, [dv * HEADS + h], ex)
            pltpu.sync_copy(exch, ex3.at[pl.ds(base * HEADS, CHUNK * HEADS)])
            return carry

        lax.fori_loop(0, CPT, chunk, 0)
        pltpu.sync_copy(dnv, dnp.at[pl.ds(t * NW4, NW4)])
        plsc.subcore_barrier()
        for ss in range(SLICE4 // RED):
            off = t * SLICE4 + ss * RED
            pltpu.sync_copy(dnp.at[pl.ds(off, RED)], slcb)

            def red(tt, carry):
                pltpu.sync_copy(dnp.at[pl.ds(tt * NW4 + off, RED)], tmpb)

                def vec(i, carry2):
                    slcb[pl.ds(i * 16, 16)] = (slcb[pl.ds(i * 16, 16)]
                                               + tmpb[pl.ds(i * 16, 16)])
                    return carry2

                lax.fori_loop(0, RED // 16, vec, 0)
                return carry

            lax.fori_loop(1, NT, red, 0)
            pltpu.sync_copy(slcb, dnf.at[pl.ds(off, RED)])
        plsc.subcore_barrier()
        pltpu.sync_copy(dnf, dnv)

        def chunk2(g, carry):
            gc = t * CPT + g
            base = gc * CHUNK
            pltpu.sync_copy(dst1d.at[pl.ds(base, CHUNK)], didxv)
            pltpu.async_copy(ex3.at[pl.ds(base * HEADS, CHUNK * HEADS)],
                             exch, sem).wait()
            for sub in range(CHUNK // 16):
                eids = lax.iota(jnp.int32, 16) + (sub * 16)
                dv = didxv[pl.ds(sub * 16, 16)]
                in_lo = dv < MID
                for h in range(HEADS):
                    exv = plsc.load_gather(exch, [eids * HEADS + h])
                    dnvv = plsc.load_gather(dnv, [dv * HEADS + h])
                    al = 0.25 * exv / (dnvv + 1e-16)
                    allo = jnp.where(in_lo, al, 0.0)
                    plsc.store_scatter(alb_lo, [eids * HEADS + h], allo)
                    plsc.store_scatter(alb_hi, [eids * HEADS + h], al - allo)
            pltpu.sync_copy(alb_lo,
                            al_lo.at[pl.ds(base * HEADS, CHUNK * HEADS)])
            pltpu.sync_copy(alb_hi,
                            al_hi.at[pl.ds(base * HEADS, CHUNK * HEADS)])
            return carry

        lax.fori_loop(0, CPT, chunk2, 0)


@functools.cache
def _sca():
    return pl.kernel(
        _sca_body,
        out_type=(
            jax.ShapeDtypeStruct((ETP * HEADS,), jnp.float32),
            jax.ShapeDtypeStruct((ETP * HEADS,), jnp.float32),
            jax.ShapeDtypeStruct((ETP * HEADS,), jnp.float32),
            jax.ShapeDtypeStruct((NT * NW4,), jnp.float32),
            jax.ShapeDtypeStruct((NW4,), jnp.float32),
        ),
        mesh=_mesh(),
        compiler_params=pltpu.CompilerParams(needs_layout_passes=False),
        scratch_types=[
            pltpu.VMEM((CHUNK,), jnp.int32),
            pltpu.VMEM((CHUNK,), jnp.int32),
            pltpu.VMEM((N * HEADS,), jnp.float32),
            pltpu.VMEM((N * HEADS,), jnp.float32),
            pltpu.VMEM((NW4,), jnp.float32),
            pltpu.VMEM((RED,), jnp.float32),
            pltpu.VMEM((RED,), jnp.float32),
            pltpu.VMEM((CHUNK * HEADS,), jnp.float32),
            pltpu.VMEM((CHUNK * HEADS,), jnp.float32),
            pltpu.VMEM((CHUNK * HEADS,), jnp.float32),
            pltpu.SemaphoreType.DMA,
        ],
    )


def _lane_splat(vec, lane):
    idx = jnp.full((16, 1), lane, jnp.int32)
    return lax.gather(
        vec, idx,
        lax.GatherDimensionNumbers(offset_dims=(), collapsed_slice_dims=(0,),
                                   start_index_map=(0,)),
        (1,), mode=lax.GatherScatterMode.PROMISE_IN_BOUNDS)


BCH = 64
BCPT = CPT * 2
NCH64 = NT * BCPT

def _scb_body(src3, dst3, al_lo, al_hi, xps, z128, out2,
              sidxa, sidxb, didxa, didxb, didxl, alba, albb, rowsa, rowsb,
              msg, accsh, sema_r, semb_r, sema_a, semb_a,
              sema_s, semb_s, sema_d, semb_d):
    c = lax.axis_index("c")
    t = lax.axis_index("s")

    for p, al3 in ((0, al_lo), (1, al_hi)):
        @pl.when(t < 8)
        def _():
            pltpu.sync_copy(z128.at[pl.ds(t * HROWS, HROWS)],
                            accsh.at[pl.ds(t * HROWS, HROWS)])
        plsc.subcore_barrier()

        def idx_load(g, sidx, didx, sem_s, sem_d):
            gc = t * BCPT + g
            pltpu.async_copy(src3.at[gc], sidx, sem_s)
            pltpu.async_copy(dst3.at[gc], didx, sem_d)

        def idx_wait(g, sidx, didx, sem_s, sem_d):
            gc = t * BCPT + g
            pltpu.make_async_copy(src3.at[gc], sidx, sem_s).wait()
            pltpu.make_async_copy(dst3.at[gc], didx, sem_d).wait()

        def rows_load(g, sidx, alb, rows, sem_r, sem_a):
            gc = t * BCPT + g
            pltpu.async_copy(al3.at[pl.ds(gc * BCH * HEADS, BCH * HEADS)],
                             alb, sem_a)
            pltpu.async_copy(xps.at[c].at[sidx.at[0]], rows, sem_r)

        def rows_wait(g, sidx, alb, rows, sem_r, sem_a):
            gc = t * BCPT + g
            pltpu.make_async_copy(
                al3.at[pl.ds(gc * BCH * HEADS, BCH * HEADS)], alb,
                sem_a).wait()
            pltpu.make_async_copy(
                xps.at[c].at[sidx.at[0]], rows, sem_r).wait()

        def make_didxl(didx):
            for k in range(BCH // 16):
                dv = didx[0, pl.ds(k * 16, 16)]
                didxl[pl.ds(k * 16, 16)] = jnp.clip(dv - p * MID, 0, MID - 1)

        def compute(alb, rows):
            def edge(e, carry2):
                eb = jnp.broadcast_to(e * HEADS, (16,))
                a0 = plsc.load_gather(alb, [eb])
                a1 = plsc.load_gather(alb, [eb + 1])
                a2 = plsc.load_gather(alb, [eb + 2])
                a3 = plsc.load_gather(alb, [eb + 3])
                for v in range(8):
                    mv = (a0 * rows[e, pl.ds(v * 16, 16)]
                          + a1 * rows[e, pl.ds(128 + v * 16, 16)]
                          + a2 * rows[e, pl.ds(256 + v * 16, 16)]
                          + a3 * rows[e, pl.ds(384 + v * 16, 16)])
                    msg[e, pl.ds(v * 16, 16)] = mv
                return carry2

            lax.fori_loop(0, BCH, edge, 0, unroll=2)
            pltpu.sync_copy(msg, accsh.at[didxl], add=True)

        bufA = (sidxa, didxa, alba, rowsa, sema_r, sema_a, sema_s, sema_d)
        bufB = (sidxb, didxb, albb, rowsb, semb_r, semb_a, semb_s, semb_d)

        def step(g, cur, nxt):
            (csi, cdi, cal, crw, csr, csa, css, csd) = cur
            (nsi, ndi, nal, nrw, nsr, nsa, nss, nsd) = nxt
            rows_wait(g, csi, cal, crw, csr, csa)
            make_didxl(cdi)

            @pl.when(g + 2 < BCPT)
            def _():
                idx_load(g + 2, csi, cdi, css, csd)

            @pl.when(g + 1 < BCPT)
            def _():
                idx_wait(g + 1, nsi, ndi, nss, nsd)
                rows_load(g + 1, nsi, nal, nrw, nsr, nsa)

            compute(cal, crw)

        def chunk(g, carry):
            @pl.when(g % 2 == 0)
            def _():
                step(g, bufA, bufB)

            @pl.when(g % 2 == 1)
            def _():
                step(g, bufB, bufA)

            return carry

        idx_load(0, sidxa, didxa, sema_s, sema_d)
        idx_load(1, sidxb, didxb, semb_s, semb_d)
        idx_wait(0, sidxa, didxa, sema_s, sema_d)
        rows_load(0, sidxa, alba, rowsa, sema_r, sema_a)
        lax.fori_loop(0, BCPT, chunk, 0)
        plsc.subcore_barrier()

        @pl.when(t < 8)
        def _():
            pltpu.sync_copy(accsh.at[pl.ds(t * HROWS, HROWS)],
                            out2.at[c].at[pl.ds(p * MID + t * HROWS, HROWS)])
        plsc.subcore_barrier()


@functools.cache
def _scb():
    return pl.kernel(
        _scb_body,
        out_type=jax.ShapeDtypeStruct((NC, NPAD, 128), jnp.float32),
        mesh=_mesh(),
        compiler_params=pltpu.CompilerParams(needs_layout_passes=False),
        scratch_types=[
            pltpu.VMEM((1, BCH), jnp.int32),
            pltpu.VMEM((1, BCH), jnp.int32),
            pltpu.VMEM((1, BCH), jnp.int32),
            pltpu.VMEM((1, BCH), jnp.int32),
            pltpu.VMEM((BCH,), jnp.int32),
            pltpu.VMEM((BCH * HEADS,), jnp.float32),
            pltpu.VMEM((BCH * HEADS,), jnp.float32),
            pltpu.VMEM((BCH, DLO), jnp.float32),
            pltpu.VMEM((BCH, DLO), jnp.float32),
            pltpu.VMEM((BCH, 128), jnp.float32),
            pltpu.VMEM_SHARED((MID, 128), jnp.float32),
            pltpu.SemaphoreType.DMA,
            pltpu.SemaphoreType.DMA,
            pltpu.SemaphoreType.DMA,
            pltpu.SemaphoreType.DMA,
            pltpu.SemaphoreType.DMA,
            pltpu.SemaphoreType.DMA,
            pltpu.SemaphoreType.DMA,
            pltpu.SemaphoreType.DMA,
        ],
    )


def _tc2_body(y0_ref, y1_ref, b_ref, g_ref, be_ref, o_ref):
    y = jnp.concatenate([y0_ref[...], y1_ref[...]], axis=1) + b_ref[...]
    mu = jnp.mean(y, axis=0, keepdims=True)
    var = jnp.mean(y * y, axis=0, keepdims=True) - mu * mu
    yn = (y - mu) * lax.rsqrt(var + 1e-5) * g_ref[...] + be_ref[...]
    o_ref[...] = jnp.maximum(yn, 0.0)


def _tc2(y0, y1, b, g, be):
    return pl.pallas_call(
        _tc2_body,
        out_shape=jax.ShapeDtypeStruct((N, OUT_DIM), jnp.float32),
    )(y0, y1, b, g, be)


def kernel(x, edge_index, W, att_src, att_dst, bias, gamma, beta):
    loops = jnp.arange(N, dtype=jnp.int32)
    src = jnp.concatenate([edge_index[0].astype(jnp.int32), loops])
    dst = jnp.concatenate([edge_index[1].astype(jnp.int32), loops])
    pad = ETP - ET
    src1d = jnp.pad(src, (0, pad))
    dst1d = jnp.pad(dst, (0, pad))

    W4 = W.reshape(IN_DIM, HEADS, 2, 128)
    wlo = W4[:, :, 0, :].reshape(IN_DIM, DLO)
    whi = W4[:, :, 1, :].reshape(IN_DIM, DLO)
    asl = att_src[:, :128].reshape(1, DLO)
    ash = att_src[:, 128:].reshape(1, DLO)
    adl = att_dst[:, :128].reshape(1, DLO)
    adh = att_dst[:, 128:].reshape(1, DLO)

    xps, asrc16, adst16 = _tc1(x, wlo, whi, asl, ash, adl, adh)
    asrc4 = asrc16[:, :HEADS].reshape(N * HEADS)
    adst4 = adst16[:, :HEADS].reshape(N * HEADS)
    zf4 = jnp.zeros((NPAD * HEADS,), jnp.float32)
    ex3, al_lo, al_hi, dnp, dnf = _sca()(src1d, dst1d, asrc4, adst4, zf4)
    del ex3, dnp, dnf
    src3 = jnp.pad(src, (0, pad)).reshape(NCH64, 1, BCH)
    dst3 = jnp.pad(dst, (0, pad)).reshape(NCH64, 1, BCH)
    z128 = jnp.zeros((NPAD, 128), jnp.float32)
    out2 = _scb()(src3, dst3, al_lo, al_hi, xps, z128)
    return _tc2(out2[0, :N], out2[1, :N], bias.reshape(1, OUT_DIM),
                gamma.reshape(1, OUT_DIM), beta.reshape(1, OUT_DIM))
